# baseline (speedup 1.0000x reference)
import jax
import jax.numpy as jnp
from jax import lax
from jax.experimental import pallas as pl
from jax.experimental.pallas import tpu as pltpu
from jax.experimental.pallas import tpu_sc as plsc

_CHUNK = 320
_NBUF = 2


def kernel(x, table):
    B, S = x.shape
    V, D = table.shape
    N = B * S
    idx = x.T.reshape(1, N)
    mesh = plsc.VectorSubcoreMesh(core_axis_name="core", subcore_axis_name="subcore")
    nw = 32
    b_per_w = N // nw
    nch = b_per_w // _CHUNK

    @pl.kernel(
        out_type=jax.ShapeDtypeStruct((N, D), table.dtype),
        mesh=mesh,
        scratch_types=[
            pltpu.VMEM((b_per_w,), jnp.int32),
            pltpu.VMEM((_NBUF, _CHUNK, D), jnp.float32),
            pltpu.SemaphoreType.DMA((_NBUF,)),
            pltpu.SemaphoreType.DMA((_NBUF,)),
        ],
    )
    def gather_kernel(table_hbm, i_hbm, o_hbm, idx_v, rows_v, gsem, ssem):
        wid = lax.axis_index("subcore") * 2 + lax.axis_index("core")
        base = wid * b_per_w
        pltpu.sync_copy(i_hbm.at[0, pl.ds(base, b_per_w)], idx_v)

        def gather_copy(c, b):
            return pltpu.make_async_copy(
                table_hbm.at[idx_v.at[pl.ds(c * _CHUNK, _CHUNK)]],
                rows_v.at[b],
                gsem.at[b],
            )

        def store_copy(c, b):
            return pltpu.make_async_copy(
                rows_v.at[b],
                o_hbm.at[pl.ds(base + c * _CHUNK, _CHUNK)],
                ssem.at[b],
            )

        for b in range(_NBUF):
            gather_copy(b, b).start()

        @pl.loop(0, nch - _NBUF, step=_NBUF)
        def _(c):
            for b in range(_NBUF):
                gather_copy(c + b, b).wait()
                store_copy(c + b, b).start()
            for b in range(_NBUF):
                store_copy(c + b, b).wait()
                gather_copy(c + _NBUF + b, b).start()

        for b in range(_NBUF):
            gather_copy(nch - _NBUF + b, b).wait()
            store_copy(nch - _NBUF + b, b).start()
        for b in range(_NBUF):
            store_copy(nch - _NBUF + b, b).wait()

    out2d = gather_kernel(table, idx)
    return out2d.reshape(S, B, D).transpose(1, 0, 2)


# speedup vs baseline: 1.0799x; 1.0799x over previous
import jax
import jax.numpy as jnp
from jax.experimental import pallas as pl
from jax.experimental.pallas import tpu as pltpu
from jax.experimental.pallas import tpu_sc as plsc

_WINDOW = 256
_STREAMS = 2


def kernel(x, table):
    B, S = x.shape
    V, D = table.shape
    N = B * S
    idx = x.T.reshape(1, N)
    mesh = plsc.VectorSubcoreMesh(core_axis_name="core", subcore_axis_name="subcore")
    sub = _WINDOW // _STREAMS

    @pl.kernel(
        out_type=jax.ShapeDtypeStruct((N, D), table.dtype),
        mesh=mesh,
        scratch_types=[pltpu.SemaphoreType.DMA((_STREAMS,))],
    )
    def gather_kernel(table_hbm, i_hbm, o_hbm, sems):
        def body(i_vmem, o_vmem):
            copies = [
                pltpu.async_copy(
                    table_hbm.at[i_vmem.at[0, pl.ds(k * sub, sub)]],
                    o_vmem.at[pl.ds(k * sub, sub)],
                    sems.at[k],
                )
                for k in range(_STREAMS)
            ]
            for c in copies:
                c.wait()

        pltpu.emit_pipeline(
            body,
            grid=(N // _WINDOW,),
            in_specs=[pl.BlockSpec((1, _WINDOW), index_map=lambda i: (0, i))],
            out_specs=[pl.BlockSpec((_WINDOW, D), index_map=lambda i: (i, 0))],
            core_axis_name=("core", "subcore"),
            dimension_semantics=(pltpu.PARALLEL,),
        )(i_hbm, o_hbm)

    out2d = gather_kernel(table, idx)
    return out2d.reshape(S, B, D).transpose(1, 0, 2)
